# baseline (speedup 1.0000x reference)
import functools

import jax
import jax.numpy as jnp
from jax.experimental import pallas as pl
from jax.experimental.pallas import tpu as pltpu


def _fold_bn(gamma, beta, mean, var, eps=1e-5):
    scale = gamma / jnp.sqrt(var + eps)
    bias = beta - mean * scale
    return scale, bias


def _dc_kernel(x_ref, m_ref, w1_ref, b1_ref, w2_ref, b2_ref, w3_ref, b3_ref,
               o_ref, col1_ref, col2_ref, h1_ref, acc_ref,
               *, S, WS, cin, cout, HW):
    f32 = jnp.float32
    bf16 = jnp.bfloat16

    def shifted(src, ky, kx):
        sh = (WS + 1 - (ky * WS + kx)) % S
        return src if sh == 0 else pltpu.roll(src, sh, axis=1)

    taps = [(ky, kx) for ky in range(3) for kx in range(3)]

    for t, (ky, kx) in enumerate(taps):
        col1_ref[t * cin:(t + 1) * cin, :] = shifted(x_ref[0], ky, kx)
    acc1 = jnp.dot(w1_ref[...], col1_ref[...], preferred_element_type=f32)
    h1_ref[...] = (jnp.maximum(acc1 + b1_ref[...], 0.0) * m_ref[...]).astype(bf16)

    for t, (ky, kx) in enumerate(taps):
        col2_ref[t * cout:(t + 1) * cout, :] = shifted(h1_ref[...], ky, kx)
    acc_ref[...] = (jnp.dot(w2_ref[...], col2_ref[...],
                            preferred_element_type=f32) + b2_ref[...])

    y = jnp.dot(w3_ref[...], acc_ref[...].astype(bf16),
                preferred_element_type=f32)
    out = jnp.maximum(acc_ref[...] + y + b3_ref[...], 0.0)
    o_ref[0] = out[:, :HW]


def kernel(x, w1, bn1_gamma, bn1_beta, bn1_mean, bn1_var,
           w2, bn2_gamma, bn2_beta, bn2_mean, bn2_var,
           w3, b3, bn3_gamma, bn3_beta, bn3_mean, bn3_var):
    N, Cin, H, W = x.shape
    Cout = w1.shape[-1]
    WS = 64
    S = (H + 2) * WS
    HW = H * WS
    f32 = jnp.float32
    bf16 = jnp.bfloat16

    s1, a1 = _fold_bn(bn1_gamma, bn1_beta, bn1_mean, bn1_var)
    s2, a2 = _fold_bn(bn2_gamma, bn2_beta, bn2_mean, bn2_var)
    s3, a3 = _fold_bn(bn3_gamma, bn3_beta, bn3_mean, bn3_var)

    w1t = (w1 * s1).reshape(9 * Cin, Cout).T.astype(bf16)
    w2t = (w2 * s2).reshape(9 * Cout, Cout).T.astype(bf16)
    w3t = (w3 * s3[None, :]).T.astype(bf16)
    b1c = a1.reshape(Cout, 1).astype(f32)
    b2c = a2.reshape(Cout, 1).astype(f32)
    b3c = (b3 * s3 + a3).reshape(Cout, 1).astype(f32)

    xp = jnp.pad(x, ((0, 0), (0, 0), (0, 2), (0, WS - W))
                 ).astype(bf16).reshape(N, Cin, S)

    ar = jnp.arange(S, dtype=jnp.int32)
    mask = ((ar // WS < H) & (ar % WS < W)).astype(f32).reshape(1, S)

    kfn = functools.partial(_dc_kernel, S=S, WS=WS, cin=Cin, cout=Cout, HW=HW)
    c2 = lambda n: (0, 0)

    out = pl.pallas_call(
        kfn,
        out_shape=jax.ShapeDtypeStruct((N, Cout, HW), f32),
        grid=(N,),
        in_specs=[
            pl.BlockSpec((1, Cin, S), lambda n: (n, 0, 0)),
            pl.BlockSpec((1, S), c2),
            pl.BlockSpec((Cout, 9 * Cin), c2),
            pl.BlockSpec((Cout, 1), c2),
            pl.BlockSpec((Cout, 9 * Cout), c2),
            pl.BlockSpec((Cout, 1), c2),
            pl.BlockSpec((Cout, Cout), c2),
            pl.BlockSpec((Cout, 1), c2),
        ],
        out_specs=pl.BlockSpec((1, Cout, HW), lambda n: (n, 0, 0)),
        scratch_shapes=[
            pltpu.VMEM((9 * Cin, S), bf16),
            pltpu.VMEM((9 * Cout, S), bf16),
            pltpu.VMEM((Cout, S), bf16),
            pltpu.VMEM((Cout, S), f32),
        ],
        compiler_params=pltpu.CompilerParams(
            dimension_semantics=("parallel",),
            vmem_limit_bytes=48 << 20,
        ),
    )(xp, mask, w1t, b1c, w2t, b2c, w3t, b3c)

    return out.reshape(N, Cout, H, WS)[:, :, :, :W]


# speedup vs baseline: 1.3380x; 1.3380x over previous
import functools

import jax
import jax.numpy as jnp
from jax.experimental import pallas as pl
from jax.experimental.pallas import tpu as pltpu


def _fold_bn(gamma, beta, mean, var, eps=1e-5):
    scale = gamma / jnp.sqrt(var + eps)
    bias = beta - mean * scale
    return scale, bias


def _dc_kernel(x_ref, m_ref, w1_ref, b1_ref, w2_ref, b2_ref, w3_ref, b3_ref,
               o_ref, xa_ref, col1_ref, col2_ref, h1_ref, acc_ref,
               *, H, W, WS, cin, cout):
    f32 = jnp.float32
    bf16 = jnp.bfloat16
    S = (H + 2) * WS

    xa_ref[...] = jnp.zeros_like(xa_ref)
    xs = x_ref[0]
    for r in range(H):
        xa_ref[:, r * WS:r * WS + W] = xs[:, r * W:(r + 1) * W]

    def shifted(src, ky, kx):
        sh = (WS + 1 - (ky * WS + kx)) % S
        return src if sh == 0 else pltpu.roll(src, sh, axis=1)

    taps = [(ky, kx) for ky in range(3) for kx in range(3)]

    for t, (ky, kx) in enumerate(taps):
        col1_ref[t * cin:(t + 1) * cin, :] = shifted(xa_ref[...], ky, kx)
    acc1 = jnp.dot(w1_ref[...], col1_ref[...], preferred_element_type=f32)
    h1_ref[...] = (jnp.maximum(acc1 + b1_ref[...], 0.0) * m_ref[...]).astype(bf16)

    for t, (ky, kx) in enumerate(taps):
        col2_ref[t * cout:(t + 1) * cout, :] = shifted(h1_ref[...], ky, kx)
    acc_ref[...] = (jnp.dot(w2_ref[...], col2_ref[...],
                            preferred_element_type=f32) + b2_ref[...])

    y = jax.lax.dot_general(acc_ref[...].astype(bf16), w3_ref[...],
                            (((0,), (0,)), ((), ())),
                            preferred_element_type=f32)
    out = jnp.maximum(y + b3_ref[...], 0.0)

    for r in range(H):
        o_ref[0, r] = out[r * WS:r * WS + W, :]


def kernel(x, w1, bn1_gamma, bn1_beta, bn1_mean, bn1_var,
           w2, bn2_gamma, bn2_beta, bn2_mean, bn2_var,
           w3, b3, bn3_gamma, bn3_beta, bn3_mean, bn3_var):
    N, Cin, H, W = x.shape
    Cout = w1.shape[-1]
    WS = 64
    S = (H + 2) * WS
    f32 = jnp.float32
    bf16 = jnp.bfloat16

    s1, a1 = _fold_bn(bn1_gamma, bn1_beta, bn1_mean, bn1_var)
    s2, a2 = _fold_bn(bn2_gamma, bn2_beta, bn2_mean, bn2_var)
    s3, a3 = _fold_bn(bn3_gamma, bn3_beta, bn3_mean, bn3_var)

    w1t = (w1 * s1).reshape(9 * Cin, Cout).T.astype(bf16)
    w2t = (w2 * s2).reshape(9 * Cout, Cout).T.astype(bf16)
    w3r = (w3 * s3[None, :] + jnp.eye(Cout, dtype=f32)).astype(bf16)
    b1c = a1.reshape(Cout, 1).astype(f32)
    b2c = a2.reshape(Cout, 1).astype(f32)
    b3r = (b3 * s3 + a3).reshape(1, Cout).astype(f32)

    xf = x.reshape(N, Cin, H * W).astype(bf16)

    ar = jnp.arange(S, dtype=jnp.int32)
    mask = ((ar // WS < H) & (ar % WS < W)).astype(f32).reshape(1, S)

    kfn = functools.partial(_dc_kernel, H=H, W=W, WS=WS, cin=Cin, cout=Cout)
    c2 = lambda n: (0, 0)

    out = pl.pallas_call(
        kfn,
        out_shape=jax.ShapeDtypeStruct((N, H, W, Cout), f32),
        grid=(N,),
        in_specs=[
            pl.BlockSpec((1, Cin, H * W), lambda n: (n, 0, 0)),
            pl.BlockSpec((1, S), c2),
            pl.BlockSpec((Cout, 9 * Cin), c2),
            pl.BlockSpec((Cout, 1), c2),
            pl.BlockSpec((Cout, 9 * Cout), c2),
            pl.BlockSpec((Cout, 1), c2),
            pl.BlockSpec((Cout, Cout), c2),
            pl.BlockSpec((1, Cout), c2),
        ],
        out_specs=pl.BlockSpec((1, H, W, Cout), lambda n: (n, 0, 0, 0)),
        scratch_shapes=[
            pltpu.VMEM((Cin, S), bf16),
            pltpu.VMEM((9 * Cin, S), bf16),
            pltpu.VMEM((9 * Cout, S), bf16),
            pltpu.VMEM((Cout, S), bf16),
            pltpu.VMEM((Cout, S), f32),
        ],
        compiler_params=pltpu.CompilerParams(
            dimension_semantics=("parallel",),
            vmem_limit_bytes=48 << 20,
        ),
    )(xf, mask, w1t, b1c, w2t, b2c, w3r, b3r)

    return jnp.transpose(out, (0, 3, 1, 2))
